# baseline (speedup 1.0000x reference)
import numpy as np
import jax
import jax.numpy as jnp
from jax.experimental import pallas as pl

NUM_AGENTS = 16
ACTION_DIM = 8
NUM_BELIEF = 120
HIDDEN = 32
HEADS = 4
D = HIDDEN * HEADS
BB = 1024


def _np_consts():
    p2 = np.zeros((128, 1024), np.float32)
    for s in range(16):
        for d in range(16):
            for h in range(HEADS):
                p2[s * 8 + h, s * 64 + d * 4 + h] = 1.0
                p2[d * 8 + 4 + h, s * 64 + d * 4 + h] = 1.0
    p20 = np.zeros((128, 64), np.float32)
    for s in range(16):
        for h in range(HEADS):
            p20[s * 8 + h, s * 4 + h] = 1.0
            p20[4 + h, s * 4 + h] += 1.0
    rsum = np.zeros((1024, 64), np.float32)
    for s in range(16):
        for d in range(16):
            for h in range(HEADS):
                rsum[s * 64 + d * 4 + h, d * 4 + h] = 1.0
    q = np.zeros((64, 2048), np.float32)
    for d in range(16):
        for h in range(HEADS):
            for c in range(HIDDEN):
                q[d * 4 + h, d * 128 + h * 32 + c] = 1.0
    r4 = np.zeros((64, 4), np.float32)
    for s in range(16):
        for h in range(HEADS):
            r4[s * 4 + h, h] = 1.0
    q4 = np.zeros((4, 128), np.float32)
    for h in range(HEADS):
        for c in range(HIDDEN):
            q4[h, h * 32 + c] = 1.0
    return p2, p20, rsum, q, r4, q4


_P2, _P20, _RSUM, _Q, _R4, _Q4 = _np_consts()


def _mm(a, b):
    return jnp.dot(a, b, preferred_element_type=jnp.float32)


def _body(sig_ref, act_ref, w1_ref, asd1_ref, b1_ref, w2_ref, asd2_ref,
          b2_ref, wtail_ref, btail_ref, p2_ref, p20_ref, rsum_ref, q_ref,
          r4_ref, q4_ref, out_ref):
    sig = sig_ref[...]
    act = act_ref[...]
    W1 = w1_ref[...]
    q = q_ref[...]
    q4 = q4_ref[...]

    w1a = W1[NUM_BELIEF:, :]
    xs = [_mm(act[:, ACTION_DIM * j:ACTION_DIM * (j + 1)], w1a)
          for j in range(NUM_AGENTS)]
    xs[0] = xs[0] + _mm(sig, W1[:NUM_BELIEF, :])

    asad = jnp.concatenate([_mm(x, asd1_ref[...]) for x in xs], axis=1)
    E = _mm(asad, p2_ref[...])
    E = jnp.where(E >= 0, E, 0.2 * E)
    rm = jnp.max(E, axis=1, keepdims=True)
    EX = jnp.exp(E - rm)
    DEN = _mm(EX, rsum_ref[...])
    DENbc = _mm(DEN, q)
    acc = [None] * NUM_AGENTS
    for s in range(NUM_AGENTS):
        bc = _mm(EX[:, 64 * s:64 * s + 64], q)
        for d in range(NUM_AGENTS):
            term = bc[:, 128 * d:128 * d + 128] * xs[s]
            acc[d] = term if acc[d] is None else acc[d] + term
    b1 = b1_ref[...]
    hs = [jnp.maximum(acc[d] / (DENbc[:, 128 * d:128 * d + 128] + 1e-16) + b1,
                      0.0)
          for d in range(NUM_AGENTS)]

    W2 = w2_ref[...]
    xs2 = [_mm(h, W2) for h in hs]
    asad2 = jnp.concatenate([_mm(x, asd2_ref[...]) for x in xs2], axis=1)
    E0 = _mm(asad2, p20_ref[...])
    E0 = jnp.where(E0 >= 0, E0, 0.2 * E0)
    rm0 = jnp.max(E0, axis=1, keepdims=True)
    EX0 = jnp.exp(E0 - rm0)
    DEN0 = _mm(EX0, r4_ref[...])
    den0bc = _mm(DEN0, q4)
    ego = None
    for s in range(NUM_AGENTS):
        term = _mm(EX0[:, 4 * s:4 * s + 4], q4) * xs2[s]
        ego = term if ego is None else ego + term
    ego = jnp.maximum(ego / (den0bc + 1e-16) + b2_ref[...], 0.0)

    out_ref[...] = _mm(ego, wtail_ref[...]) + btail_ref[...]


def _asd(att_src, att_dst):
    src_flat = att_src.reshape(-1)
    dst_flat = att_dst.reshape(-1)
    mask = jnp.asarray(_Q4.T)
    return jnp.concatenate([mask * src_flat[:, None],
                            mask * dst_flat[:, None]], axis=1)


def kernel(signals, neighbor_actions, W1, att_src1, att_dst1, b1, W2,
           att_src2, att_dst2, b2, Wq, bq, Wk, bk, Wv, bv, Wo, bo,
           Wlm, blm, Wap, bap):
    B = signals.shape[0]
    grid = (B // BB,)
    asd1 = _asd(att_src1, att_dst1)
    asd2 = _asd(att_src2, att_dst2)
    m1 = Wlm @ Wap
    m2 = Wo @ m1
    wtail = Wv @ m2
    btail = bv @ m2 + bo @ m1 + blm @ Wap + bap
    consts = [jnp.asarray(c) for c in (_P2, _P20, _RSUM, _Q, _R4, _Q4)]

    def full(a):
        return pl.BlockSpec(a.shape, lambda i: (0,) * a.ndim)

    weights = [W1, asd1, b1.reshape(1, D), W2, asd2, b2.reshape(1, D),
               wtail, btail.reshape(1, -1)] + consts

    return pl.pallas_call(
        _body,
        grid=grid,
        in_specs=[pl.BlockSpec((BB, NUM_BELIEF), lambda i: (i, 0)),
                  pl.BlockSpec((BB, NUM_AGENTS * ACTION_DIM), lambda i: (i, 0))]
                 + [full(w) for w in weights],
        out_specs=pl.BlockSpec((BB, ACTION_DIM * NUM_AGENTS), lambda i: (i, 0)),
        out_shape=jax.ShapeDtypeStruct((B, ACTION_DIM * NUM_AGENTS),
                                       jnp.float32),
    )(signals, neighbor_actions, *weights)


# speedup vs baseline: 1.0455x; 1.0455x over previous
import numpy as np
import jax
import jax.numpy as jnp
from jax.experimental import pallas as pl

NUM_AGENTS = 16
ACTION_DIM = 8
NUM_BELIEF = 120
HIDDEN = 32
HEADS = 4
D = HIDDEN * HEADS
BB = 1024


def _np_consts():
    p2 = np.zeros((128, 1024), np.float32)
    for s in range(16):
        for d in range(16):
            for h in range(HEADS):
                p2[s * 8 + h, s * 64 + d * 4 + h] = 1.0
                p2[d * 8 + 4 + h, s * 64 + d * 4 + h] = 1.0
    p20 = np.zeros((128, 64), np.float32)
    for s in range(16):
        for h in range(HEADS):
            p20[s * 8 + h, s * 4 + h] = 1.0
            p20[4 + h, s * 4 + h] += 1.0
    rsum = np.zeros((1024, 64), np.float32)
    for s in range(16):
        for d in range(16):
            for h in range(HEADS):
                rsum[s * 64 + d * 4 + h, d * 4 + h] = 1.0
    q = np.zeros((64, 2048), np.float32)
    for d in range(16):
        for h in range(HEADS):
            for c in range(HIDDEN):
                q[d * 4 + h, d * 128 + h * 32 + c] = 1.0
    r4 = np.zeros((64, 4), np.float32)
    for s in range(16):
        for h in range(HEADS):
            r4[s * 4 + h, h] = 1.0
    q4 = np.zeros((4, 128), np.float32)
    for h in range(HEADS):
        for c in range(HIDDEN):
            q4[h, h * 32 + c] = 1.0
    return p2, p20, rsum, q, r4, q4


_P2, _P20, _RSUM, _Q, _R4, _Q4 = _np_consts()


def _mm(a, b):
    return jnp.dot(a, b, preferred_element_type=jnp.float32)


def _body(sig_ref, act_ref, w1_ref, asd1_ref, b1_ref, w2_ref, asd2_ref,
          b2_ref, wv_ref, bv_ref, wo_ref, bo_ref, wlm_ref, blm_ref,
          wap_ref, bap_ref, p2_ref, p20_ref, rsum_ref, q_ref, r4_ref,
          q4_ref, out_ref):
    sig = sig_ref[...]
    act = act_ref[...]
    W1 = w1_ref[...]
    q = q_ref[...]
    q4 = q4_ref[...]

    w1a = W1[NUM_BELIEF:, :]
    xs = [_mm(act[:, ACTION_DIM * j:ACTION_DIM * (j + 1)], w1a)
          for j in range(NUM_AGENTS)]
    xs[0] = xs[0] + _mm(sig, W1[:NUM_BELIEF, :])

    asad = jnp.concatenate([_mm(x, asd1_ref[...]) for x in xs], axis=1)
    E = _mm(asad, p2_ref[...])
    E = jnp.where(E >= 0, E, 0.2 * E)
    rm = jnp.max(E, axis=1, keepdims=True)
    EX = jnp.exp(E - rm)
    DEN = _mm(EX, rsum_ref[...])
    DENbc = _mm(DEN, q)
    acc = [None] * NUM_AGENTS
    for s in range(NUM_AGENTS):
        bc = _mm(EX[:, 64 * s:64 * s + 64], q)
        for d in range(NUM_AGENTS):
            term = bc[:, 128 * d:128 * d + 128] * xs[s]
            acc[d] = term if acc[d] is None else acc[d] + term
    b1 = b1_ref[...]
    hs = [jnp.maximum(acc[d] / (DENbc[:, 128 * d:128 * d + 128] + 1e-16) + b1,
                      0.0)
          for d in range(NUM_AGENTS)]

    W2 = w2_ref[...]
    xs2 = [_mm(h, W2) for h in hs]
    asad2 = jnp.concatenate([_mm(x, asd2_ref[...]) for x in xs2], axis=1)
    E0 = _mm(asad2, p20_ref[...])
    E0 = jnp.where(E0 >= 0, E0, 0.2 * E0)
    rm0 = jnp.max(E0, axis=1, keepdims=True)
    EX0 = jnp.exp(E0 - rm0)
    DEN0 = _mm(EX0, r4_ref[...])
    den0bc = _mm(DEN0, q4)
    ego = None
    for s in range(NUM_AGENTS):
        term = _mm(EX0[:, 4 * s:4 * s + 4], q4) * xs2[s]
        ego = term if ego is None else ego + term
    ego = jnp.maximum(ego / (den0bc + 1e-16) + b2_ref[...], 0.0)

    v = _mm(ego, wv_ref[...]) + bv_ref[...]
    tf = _mm(v, wo_ref[...]) + bo_ref[...]
    mean = _mm(tf, wlm_ref[...]) + blm_ref[...]
    out_ref[...] = _mm(mean, wap_ref[...]) + bap_ref[...]


def _asd(att_src, att_dst):
    src_flat = att_src.reshape(-1)
    dst_flat = att_dst.reshape(-1)
    mask = jnp.asarray(_Q4.T)
    return jnp.concatenate([mask * src_flat[:, None],
                            mask * dst_flat[:, None]], axis=1)


def kernel(signals, neighbor_actions, W1, att_src1, att_dst1, b1, W2,
           att_src2, att_dst2, b2, Wq, bq, Wk, bk, Wv, bv, Wo, bo,
           Wlm, blm, Wap, bap):
    B = signals.shape[0]
    grid = (B // BB,)
    asd1 = _asd(att_src1, att_dst1)
    asd2 = _asd(att_src2, att_dst2)
    consts = [jnp.asarray(c) for c in (_P2, _P20, _RSUM, _Q, _R4, _Q4)]

    def full(a):
        return pl.BlockSpec(a.shape, lambda i: (0,) * a.ndim)

    weights = [W1, asd1, b1.reshape(1, D), W2, asd2, b2.reshape(1, D),
               Wv, bv.reshape(1, D), Wo, bo.reshape(1, D),
               Wlm, blm.reshape(1, -1), Wap, bap.reshape(1, -1)] + consts

    return pl.pallas_call(
        _body,
        grid=grid,
        in_specs=[pl.BlockSpec((BB, NUM_BELIEF), lambda i: (i, 0)),
                  pl.BlockSpec((BB, NUM_AGENTS * ACTION_DIM), lambda i: (i, 0))]
                 + [full(w) for w in weights],
        out_specs=pl.BlockSpec((BB, ACTION_DIM * NUM_AGENTS), lambda i: (i, 0)),
        out_shape=jax.ShapeDtypeStruct((B, ACTION_DIM * NUM_AGENTS),
                                       jnp.float32),
    )(signals, neighbor_actions, *weights)
